# baseline (speedup 1.0000x reference)
import functools

import jax
import jax.numpy as jnp
from jax import lax
from jax.experimental import pallas as pl
from jax.experimental.pallas import tpu as pltpu
from jax.experimental.pallas import tpu_sc as plsc

KNN = 8
M_CH = 32
F2 = 20
EPS = 1e-5
SLOPE = 0.2
DPAD = 64

QBL = 256
SCH = 1024
QB3 = 512
R3 = QB3 * KNN

INF = float("inf")
IMAX = 2**31 - 1


def _argmin_tree(d, i):
    n = d.shape[0]
    while n > 1:
        h = n // 2
        cmp = d[:h] <= d[h:]
        d = jnp.minimum(d[:h], d[h:])
        i = jnp.where(cmp, i[:h], i[h:])
        n = h
    return d, i


def _topk_rows(d, i, k):
    vs, ids = [], []
    for _ in range(k):
        v, im = _argmin_tree(d, i)
        vs.append(v)
        ids.append(im)
        d = jnp.where(i == im, INF, d)
    return jnp.concatenate(vs, axis=0), jnp.concatenate(ids, axis=0)


def _knn_body(q_ref, s_ref, o_ref, bd_ref, bi_ref):
    c = pl.program_id(1)
    n_c = pl.num_programs(1)

    @pl.when(c == 0)
    def _():
        bd_ref[...] = jnp.full((KNN, QBL), INF, jnp.float32)
        bi_ref[...] = jnp.full((KNN, QBL), IMAX, jnp.int32)

    q = q_ref[...]
    s = s_ref[...]
    dx = s[:, 0:1] - q[0:1, :]
    dy = s[:, 1:2] - q[1:2, :]
    dz = s[:, 2:3] - q[2:3, :]
    d2 = (dx * dx + dy * dy) + dz * dz
    ii = lax.broadcasted_iota(jnp.int32, (SCH, QBL), 0)

    cd, ci = _topk_rows(d2, ii, KNN)
    ci = ci + c * SCH

    comb_d = jnp.concatenate([bd_ref[...], cd], axis=0)
    comb_i = jnp.concatenate([bi_ref[...], ci], axis=0)
    nd, ni = _topk_rows(comb_d, comb_i, KNN)
    bd_ref[...] = nd
    bi_ref[...] = ni

    @pl.when(c == n_c - 1)
    def _():
        bi = bi_ref[...]
        first = jnp.broadcast_to(bi[0:1, :], (KNN, QBL))
        o_ref[...] = jnp.where(bi == 0, first, bi)


def _knn_topk(xyz2_t, xyz1, interpret=False):
    n_q = xyz2_t.shape[1]
    n_s = xyz1.shape[0]
    grid = (n_q // QBL, n_s // SCH)
    return pl.pallas_call(
        _knn_body,
        grid=grid,
        in_specs=[
            pl.BlockSpec((3, QBL), lambda i, c: (0, i)),
            pl.BlockSpec((SCH, 3), lambda i, c: (c, 0)),
        ],
        out_specs=pl.BlockSpec((KNN, QBL), lambda i, c: (0, i)),
        out_shape=jax.ShapeDtypeStruct((KNN, n_q), jnp.int32),
        scratch_shapes=[
            pltpu.VMEM((KNN, QBL), jnp.float32),
            pltpu.VMEM((KNN, QBL), jnp.int32),
        ],
        compiler_params=pltpu.CompilerParams(
            dimension_semantics=("arbitrary", "arbitrary"),
        ),
        interpret=interpret,
    )(xyz2_t, xyz1)


def _sc_gather(table, idx):
    info = plsc.get_sparse_core_info()
    nw = info.num_cores * info.num_subcores
    b = idx.shape[0]
    dp = table.shape[1]
    b_per_w = b // nw
    ch = 1024
    n_ch = b_per_w // ch
    mesh = plsc.VectorSubcoreMesh(core_axis_name="c", subcore_axis_name="s")

    @functools.partial(
        pl.kernel,
        mesh=mesh,
        out_type=jax.ShapeDtypeStruct((b, dp), jnp.float32),
        scratch_types=[
            pltpu.VMEM((ch,), jnp.int32),
            pltpu.VMEM((ch, dp), jnp.float32),
            pltpu.SemaphoreType.DMA,
        ],
        compiler_params=pltpu.CompilerParams(use_tc_tiling_on_sc=False),
    )
    def k(table_hbm, idx_hbm, out_hbm, idx_v, rows_v, sem):
        wid = lax.axis_index("s") * info.num_cores + lax.axis_index("c")
        base = wid * b_per_w

        def body(j, carry):
            off = base + j * ch
            pltpu.sync_copy(idx_hbm.at[pl.ds(off, ch)], idx_v)
            pltpu.async_copy(table_hbm.at[idx_v], rows_v, sem).wait()
            pltpu.sync_copy(rows_v, out_hbm.at[pl.ds(off, ch)])
            return carry

        lax.fori_loop(0, n_ch, body, 0)

    return k(table, idx)


def _gw_body(g_ref, cent_ref, wrt_ref, br_ref, gw_ref, st_ref):
    i = pl.program_id(0)
    g = g_ref[...]
    gx = g[:, 0:3]
    c = cent_ref[...]
    off = gx - c
    ox, oy, oz = off[:, 0:1], off[:, 1:2], off[:, 2:3]
    dist = jnp.sqrt((ox * ox + oy * oy) + oz * oz)
    rel = jnp.concatenate([off, c, gx, dist], axis=1)
    gw = jnp.dot(rel, wrt_ref[...], preferred_element_type=jnp.float32)
    gw = gw + br_ref[...]
    gw_ref[...] = gw

    @pl.when(i == 0)
    def _():
        st_ref[...] = jnp.zeros_like(st_ref)

    s1 = jnp.sum(gw, axis=0, keepdims=True)
    s2 = jnp.sum(gw * gw, axis=0, keepdims=True)
    st_ref[...] += jnp.concatenate([s1, s2], axis=0)


def _gw_pass(g2d, cent, wrt, br_row, interpret=False):
    n_rows = g2d.shape[0]
    grid = (n_rows // R3,)
    return pl.pallas_call(
        _gw_body,
        grid=grid,
        in_specs=[
            pl.BlockSpec((R3, DPAD), lambda i: (i, 0)),
            pl.BlockSpec((R3, 3), lambda i: (i, 0)),
            pl.BlockSpec((10, M_CH), lambda i: (0, 0)),
            pl.BlockSpec((1, M_CH), lambda i: (0, 0)),
        ],
        out_specs=[
            pl.BlockSpec((R3, M_CH), lambda i: (i, 0)),
            pl.BlockSpec((2, M_CH), lambda i: (0, 0)),
        ],
        out_shape=[
            jax.ShapeDtypeStruct((n_rows, M_CH), jnp.float32),
            jax.ShapeDtypeStruct((2, M_CH), jnp.float32),
        ],
        compiler_params=pltpu.CompilerParams(
            dimension_semantics=("arbitrary",),
        ),
        interpret=interpret,
    )(g2d, cent, wrt, br_row)


def _fuse_body(g_ref, gw_ref, a_ref, b_ref, f2_ref, wft_ref, bf_ref, o_ref, st_ref):
    i = pl.program_id(0)
    gfeat = g_ref[...][:, 3:3 + M_CH]
    w = gw_ref[...] * a_ref[...] + b_ref[...]
    w = jnp.where(w >= 0, w, SLOPE * w)
    weighted = gfeat * w
    upd = jnp.mean(weighted.reshape(QB3, KNN, M_CH), axis=1)
    fused = jnp.concatenate([upd, f2_ref[...]], axis=1)
    out = jnp.dot(fused, wft_ref[...], preferred_element_type=jnp.float32)
    out = out + bf_ref[...]
    o_ref[...] = out

    @pl.when(i == 0)
    def _():
        st_ref[...] = jnp.zeros_like(st_ref)

    s1 = jnp.sum(out, axis=0, keepdims=True)
    s2 = jnp.sum(out * out, axis=0, keepdims=True)
    st_ref[...] += jnp.concatenate([s1, s2], axis=0)


def _fuse_pass(g2d, gw, a_row, b_row, features2, wft, bf_row, interpret=False):
    n_rows = g2d.shape[0]
    n_q = features2.shape[0]
    grid = (n_rows // R3,)
    return pl.pallas_call(
        _fuse_body,
        grid=grid,
        in_specs=[
            pl.BlockSpec((R3, DPAD), lambda i: (i, 0)),
            pl.BlockSpec((R3, M_CH), lambda i: (i, 0)),
            pl.BlockSpec((1, M_CH), lambda i: (0, 0)),
            pl.BlockSpec((1, M_CH), lambda i: (0, 0)),
            pl.BlockSpec((QB3, F2), lambda i: (i, 0)),
            pl.BlockSpec((M_CH + F2, F2), lambda i: (0, 0)),
            pl.BlockSpec((1, F2), lambda i: (0, 0)),
        ],
        out_specs=[
            pl.BlockSpec((QB3, F2), lambda i: (i, 0)),
            pl.BlockSpec((2, F2), lambda i: (0, 0)),
        ],
        out_shape=[
            jax.ShapeDtypeStruct((n_q, F2), jnp.float32),
            jax.ShapeDtypeStruct((2, F2), jnp.float32),
        ],
        compiler_params=pltpu.CompilerParams(
            dimension_semantics=("arbitrary",),
        ),
        interpret=interpret,
    )(g2d, gw, a_row, b_row, features2, wft, bf_row)


def _bn2_body(x_ref, a_ref, b_ref, o_ref):
    y = x_ref[...] * a_ref[...] + b_ref[...]
    o_ref[...] = jnp.where(y >= 0, y, SLOPE * y)


def _bn2_pass(x, a_row, b_row, interpret=False):
    n_q = x.shape[0]
    blk = min(2048, n_q)
    grid = (n_q // blk,)
    return pl.pallas_call(
        _bn2_body,
        grid=grid,
        in_specs=[
            pl.BlockSpec((blk, F2), lambda i: (i, 0)),
            pl.BlockSpec((1, F2), lambda i: (0, 0)),
            pl.BlockSpec((1, F2), lambda i: (0, 0)),
        ],
        out_specs=pl.BlockSpec((blk, F2), lambda i: (i, 0)),
        out_shape=jax.ShapeDtypeStruct((n_q, F2), jnp.float32),
        interpret=interpret,
    )(x, a_row, b_row)


def kernel(xyz1, features1, xyz2, features2, Wr, br, gr, betar, Wf, bf, gf, betaf):
    n_s = xyz1.shape[0]
    n_q = xyz2.shape[0]

    ind = _knn_topk(xyz2.T, xyz1).T

    pad = jnp.zeros((n_s, DPAD - 3 - M_CH), jnp.float32)
    table = jnp.concatenate([xyz1, features1, pad], axis=1)
    g2d = _sc_gather(table, ind.reshape(-1))

    cent = jnp.repeat(xyz2, KNN, axis=0)
    gw, st1 = _gw_pass(g2d, cent, Wr.T, br.reshape(1, -1))

    n1 = jnp.float32(n_q * KNN)
    mu1 = st1[0] / n1
    var1 = st1[1] / n1 - mu1 * mu1
    a1 = gr / jnp.sqrt(var1 + EPS)
    b1 = betar - mu1 * a1

    out1, st2 = _fuse_pass(
        g2d, gw, a1.reshape(1, -1), b1.reshape(1, -1), features2, Wf.T,
        bf.reshape(1, -1))

    n2 = jnp.float32(n_q)
    mu2 = st2[0] / n2
    var2 = st2[1] / n2 - mu2 * mu2
    a2 = gf / jnp.sqrt(var2 + EPS)
    b2 = betaf - mu2 * a2

    return _bn2_pass(out1, a2.reshape(1, -1), b2.reshape(1, -1))


# speedup vs baseline: 1.0383x; 1.0383x over previous
import functools

import jax
import jax.numpy as jnp
from jax import lax
from jax.experimental import pallas as pl
from jax.experimental.pallas import tpu as pltpu
from jax.experimental.pallas import tpu_sc as plsc

KNN = 8
M_CH = 32
F2 = 20
EPS = 1e-5
SLOPE = 0.2
DPAD = 64

QBL = 256
SCH = 512
QB3 = 512
R3 = QB3 * KNN

INF = float("inf")
IMAX = 2**31 - 1


def _argmin_tree(d, i):
    n = d.shape[0]
    while n > 1:
        h = n // 2
        cmp = d[:h] <= d[h:]
        d = jnp.minimum(d[:h], d[h:])
        i = jnp.where(cmp, i[:h], i[h:])
        n = h
    return d, i


def _topk_rows(d, i, k):
    vs, ids = [], []
    for _ in range(k):
        v, im = _argmin_tree(d, i)
        vs.append(v)
        ids.append(im)
        d = jnp.where(i == im, INF, d)
    return jnp.concatenate(vs, axis=0), jnp.concatenate(ids, axis=0)


def _knn_body(q_ref, s_ref, o_ref, bd_ref, bi_ref):
    c = pl.program_id(1)
    n_c = pl.num_programs(1)

    @pl.when(c == 0)
    def _():
        bd_ref[...] = jnp.full((KNN, QBL), INF, jnp.float32)
        bi_ref[...] = jnp.full((KNN, QBL), IMAX, jnp.int32)

    q = q_ref[...]
    s = s_ref[...]
    dx = s[:, 0:1] - q[0:1, :]
    dy = s[:, 1:2] - q[1:2, :]
    dz = s[:, 2:3] - q[2:3, :]
    d2 = (dx * dx + dy * dy) + dz * dz
    ii = lax.broadcasted_iota(jnp.int32, (SCH, QBL), 0)

    cd, ci = _topk_rows(d2, ii, KNN)
    ci = ci + c * SCH

    comb_d = jnp.concatenate([bd_ref[...], cd], axis=0)
    comb_i = jnp.concatenate([bi_ref[...], ci], axis=0)
    nd, ni = _topk_rows(comb_d, comb_i, KNN)
    bd_ref[...] = nd
    bi_ref[...] = ni

    @pl.when(c == n_c - 1)
    def _():
        bi = bi_ref[...]
        first = jnp.broadcast_to(bi[0:1, :], (KNN, QBL))
        o_ref[...] = jnp.where(bi == 0, first, bi)


def _knn_topk(xyz2_t, xyz1, interpret=False):
    n_q = xyz2_t.shape[1]
    n_s = xyz1.shape[0]
    grid = (n_q // QBL, n_s // SCH)
    return pl.pallas_call(
        _knn_body,
        grid=grid,
        in_specs=[
            pl.BlockSpec((3, QBL), lambda i, c: (0, i)),
            pl.BlockSpec((SCH, 3), lambda i, c: (c, 0)),
        ],
        out_specs=pl.BlockSpec((KNN, QBL), lambda i, c: (0, i)),
        out_shape=jax.ShapeDtypeStruct((KNN, n_q), jnp.int32),
        scratch_shapes=[
            pltpu.VMEM((KNN, QBL), jnp.float32),
            pltpu.VMEM((KNN, QBL), jnp.int32),
        ],
        compiler_params=pltpu.CompilerParams(
            dimension_semantics=("arbitrary", "arbitrary"),
        ),
        interpret=interpret,
    )(xyz2_t, xyz1)


def _sc_gather(table, idx):
    info = plsc.get_sparse_core_info()
    nw = info.num_cores * info.num_subcores
    b = idx.shape[0]
    dp = table.shape[1]
    b_per_w = b // nw
    ch = 1024
    n_ch = b_per_w // ch
    mesh = plsc.VectorSubcoreMesh(core_axis_name="c", subcore_axis_name="s")

    @functools.partial(
        pl.kernel,
        mesh=mesh,
        out_type=jax.ShapeDtypeStruct((b, dp), jnp.float32),
        scratch_types=[
            pltpu.VMEM((ch,), jnp.int32),
            pltpu.VMEM((ch, dp), jnp.float32),
            pltpu.SemaphoreType.DMA,
        ],
        compiler_params=pltpu.CompilerParams(use_tc_tiling_on_sc=False),
    )
    def k(table_hbm, idx_hbm, out_hbm, idx_v, rows_v, sem):
        wid = lax.axis_index("s") * info.num_cores + lax.axis_index("c")
        base = wid * b_per_w

        def body(j, carry):
            off = base + j * ch
            pltpu.sync_copy(idx_hbm.at[pl.ds(off, ch)], idx_v)
            pltpu.async_copy(table_hbm.at[idx_v], rows_v, sem).wait()
            pltpu.sync_copy(rows_v, out_hbm.at[pl.ds(off, ch)])
            return carry

        lax.fori_loop(0, n_ch, body, 0)

    return k(table, idx)


def _gw_body(g_ref, cent_ref, wa_ref, wb_ref, w9_ref, br_ref, gw_ref, st_ref):
    i = pl.program_id(0)
    g = g_ref[...]
    gx = g[:, 0:3]
    c = cent_ref[...]
    off = gx - c
    ox, oy, oz = off[:, 0:1], off[:, 1:2], off[:, 2:3]
    dist = jnp.sqrt((ox * ox + oy * oy) + oz * oz)
    gw = jnp.dot(gx, wa_ref[...], preferred_element_type=jnp.float32)
    gw = gw + jnp.dot(c, wb_ref[...], preferred_element_type=jnp.float32)
    gw = gw + dist * w9_ref[...]
    gw = gw + br_ref[...]
    gw_ref[...] = gw

    @pl.when(i == 0)
    def _():
        st_ref[...] = jnp.zeros_like(st_ref)

    s1 = jnp.sum(gw, axis=0, keepdims=True)
    s2 = jnp.sum(gw * gw, axis=0, keepdims=True)
    st_ref[...] += jnp.concatenate([s1, s2], axis=0)


def _gw_pass(g2d, cent, wa, wb, w9_row, br_row, interpret=False):
    n_rows = g2d.shape[0]
    grid = (n_rows // R3,)
    return pl.pallas_call(
        _gw_body,
        grid=grid,
        in_specs=[
            pl.BlockSpec((R3, DPAD), lambda i: (i, 0)),
            pl.BlockSpec((R3, 3), lambda i: (i, 0)),
            pl.BlockSpec((3, M_CH), lambda i: (0, 0)),
            pl.BlockSpec((3, M_CH), lambda i: (0, 0)),
            pl.BlockSpec((1, M_CH), lambda i: (0, 0)),
            pl.BlockSpec((1, M_CH), lambda i: (0, 0)),
        ],
        out_specs=[
            pl.BlockSpec((R3, M_CH), lambda i: (i, 0)),
            pl.BlockSpec((2, M_CH), lambda i: (0, 0)),
        ],
        out_shape=[
            jax.ShapeDtypeStruct((n_rows, M_CH), jnp.float32),
            jax.ShapeDtypeStruct((2, M_CH), jnp.float32),
        ],
        compiler_params=pltpu.CompilerParams(
            dimension_semantics=("arbitrary",),
        ),
        interpret=interpret,
    )(g2d, cent, wa, wb, w9_row, br_row)


def _fuse_body(g_ref, gw_ref, a_ref, b_ref, f2_ref, wft_ref, bf_ref, o_ref, st_ref):
    i = pl.program_id(0)
    gfeat = g_ref[...][:, 3:3 + M_CH]
    w = gw_ref[...] * a_ref[...] + b_ref[...]
    w = jnp.where(w >= 0, w, SLOPE * w)
    weighted = gfeat * w
    upd = jnp.mean(weighted.reshape(QB3, KNN, M_CH), axis=1)
    fused = jnp.concatenate([upd, f2_ref[...]], axis=1)
    out = jnp.dot(fused, wft_ref[...], preferred_element_type=jnp.float32)
    out = out + bf_ref[...]
    o_ref[...] = out

    @pl.when(i == 0)
    def _():
        st_ref[...] = jnp.zeros_like(st_ref)

    s1 = jnp.sum(out, axis=0, keepdims=True)
    s2 = jnp.sum(out * out, axis=0, keepdims=True)
    st_ref[...] += jnp.concatenate([s1, s2], axis=0)


def _fuse_pass(g2d, gw, a_row, b_row, features2, wft, bf_row, interpret=False):
    n_rows = g2d.shape[0]
    n_q = features2.shape[0]
    grid = (n_rows // R3,)
    return pl.pallas_call(
        _fuse_body,
        grid=grid,
        in_specs=[
            pl.BlockSpec((R3, DPAD), lambda i: (i, 0)),
            pl.BlockSpec((R3, M_CH), lambda i: (i, 0)),
            pl.BlockSpec((1, M_CH), lambda i: (0, 0)),
            pl.BlockSpec((1, M_CH), lambda i: (0, 0)),
            pl.BlockSpec((QB3, F2), lambda i: (i, 0)),
            pl.BlockSpec((M_CH + F2, F2), lambda i: (0, 0)),
            pl.BlockSpec((1, F2), lambda i: (0, 0)),
        ],
        out_specs=[
            pl.BlockSpec((QB3, F2), lambda i: (i, 0)),
            pl.BlockSpec((2, F2), lambda i: (0, 0)),
        ],
        out_shape=[
            jax.ShapeDtypeStruct((n_q, F2), jnp.float32),
            jax.ShapeDtypeStruct((2, F2), jnp.float32),
        ],
        compiler_params=pltpu.CompilerParams(
            dimension_semantics=("arbitrary",),
        ),
        interpret=interpret,
    )(g2d, gw, a_row, b_row, features2, wft, bf_row)


def _bn2_body(x_ref, a_ref, b_ref, o_ref):
    y = x_ref[...] * a_ref[...] + b_ref[...]
    o_ref[...] = jnp.where(y >= 0, y, SLOPE * y)


def _bn2_pass(x, a_row, b_row, interpret=False):
    n_q = x.shape[0]
    blk = min(2048, n_q)
    grid = (n_q // blk,)
    return pl.pallas_call(
        _bn2_body,
        grid=grid,
        in_specs=[
            pl.BlockSpec((blk, F2), lambda i: (i, 0)),
            pl.BlockSpec((1, F2), lambda i: (0, 0)),
            pl.BlockSpec((1, F2), lambda i: (0, 0)),
        ],
        out_specs=pl.BlockSpec((blk, F2), lambda i: (i, 0)),
        out_shape=jax.ShapeDtypeStruct((n_q, F2), jnp.float32),
        interpret=interpret,
    )(x, a_row, b_row)


def kernel(xyz1, features1, xyz2, features2, Wr, br, gr, betar, Wf, bf, gf, betaf):
    n_s = xyz1.shape[0]
    n_q = xyz2.shape[0]

    ind = _knn_topk(xyz2.T, xyz1).T

    pad = jnp.zeros((n_s, DPAD - 3 - M_CH), jnp.float32)
    table = jnp.concatenate([xyz1, features1, pad], axis=1)
    g2d = _sc_gather(table, ind.reshape(-1))

    cent = jnp.repeat(xyz2, KNN, axis=0)
    wrt = Wr.T
    wa = wrt[0:3] + wrt[6:9]
    wb = wrt[3:6] - wrt[0:3]
    gw, st1 = _gw_pass(g2d, cent, wa, wb, wrt[9:10], br.reshape(1, -1))

    n1 = jnp.float32(n_q * KNN)
    mu1 = st1[0] / n1
    var1 = st1[1] / n1 - mu1 * mu1
    a1 = gr / jnp.sqrt(var1 + EPS)
    b1 = betar - mu1 * a1

    out1, st2 = _fuse_pass(
        g2d, gw, a1.reshape(1, -1), b1.reshape(1, -1), features2, Wf.T,
        bf.reshape(1, -1))

    n2 = jnp.float32(n_q)
    mu2 = st2[0] / n2
    var2 = st2[1] / n2 - mu2 * mu2
    a2 = gf / jnp.sqrt(var2 + EPS)
    b2 = betaf - mu2 * a2

    return _bn2_pass(out1, a2.reshape(1, -1), b2.reshape(1, -1))
